# Initial kernel scaffold; baseline (speedup 1.0000x reference)
import jax
import jax.numpy as jnp
from jax.experimental import pallas as pl


def kernel(node_feat, edge_index, W1, b1, gamma1, beta1, W2, b2, Wf, bf):
    raise NotImplementedError("write your pallas kernel here")


# speedup vs baseline: 19.4018x; 19.4018x over previous
import functools

import jax
import jax.numpy as jnp
from jax import lax
from jax.experimental import pallas as pl
from jax.experimental.pallas import tpu as pltpu
from jax.experimental.pallas import tpu_sc as plsc

_N = 10000
_NP = 10240
_E = 320000
_D = 128
_CW = 16
_NC, _NS = 2, 16
_NW = _NC * _NS
_EW = _E // _NW
_K = 80
_T = _EW // _K
_RPT = _NP // _NS
_ZR = 32

assert _EW * _NW == _E and _T * _K == _EW and _RPT * _NS == _NP
assert _RPT % _ZR == 0 and _RPT % 8 == 0 and _K % 8 == 0 and _EW % 8 == 0

_mesh = plsc.VectorSubcoreMesh(core_axis_name="c", subcore_axis_name="s")


def _zero_fill(buf, rows, width, zslices, acc, row0):
    zero = jnp.zeros((16,), jnp.float32)
    for r in range(rows):
        for c in range(width // 16):
            buf[r, pl.ds(c * 16, 16)] = zero
    for z in range(zslices):
        pltpu.sync_copy(buf, acc.at[pl.ds(row0 + z * rows, rows)])


@functools.partial(
    pl.kernel,
    mesh=_mesh,
    out_type=jax.ShapeDtypeStruct((_NC, _NP, _CW), jnp.float32),
    scratch_types=[
        pltpu.VMEM((_T, _K), jnp.int32),
        pltpu.VMEM((_K, _CW), jnp.float32),
        pltpu.VMEM((_ZR, _CW), jnp.float32),
        pltpu.VMEM_SHARED((_NP, _CW), jnp.float32),
    ],
)
def _sc_degree(dst_hbm, out_hbm, idx_d, ones_b, zbuf, cnt):
    cid = lax.axis_index("c")
    sid = lax.axis_index("s")
    w = cid * _NS + sid
    row0 = sid * _RPT

    one = jnp.full((16,), 1.0, jnp.float32)
    for r in range(_K):
        ones_b[r, pl.ds(0, 16)] = one
    _zero_fill(zbuf, _ZR, _CW, _RPT // _ZR, cnt, row0)
    plsc.subcore_barrier()

    pltpu.sync_copy(dst_hbm.at[w], idx_d)

    def body(j, carry):
        pltpu.sync_copy(ones_b, cnt.at[idx_d.at[j]], add=True)
        return carry

    lax.fori_loop(0, _T, body, 0)
    plsc.subcore_barrier()
    pltpu.sync_copy(cnt.at[pl.ds(row0, _RPT)],
                    out_hbm.at[cid, pl.ds(row0, _RPT)])


@functools.partial(
    pl.kernel,
    mesh=_mesh,
    out_type=jax.ShapeDtypeStruct((_NC, _NP, _D), jnp.float32),
    scratch_types=[
        pltpu.VMEM((_T, _K), jnp.int32),
        pltpu.VMEM((_T, _K), jnp.int32),
        pltpu.VMEM((_K, _D), jnp.float32),
        pltpu.VMEM((_ZR, _D), jnp.float32),
        pltpu.VMEM_SHARED((_NP, _D), jnp.float32),
        pltpu.SemaphoreType.DMA,
    ],
)
def _sc_scatter(y_hbm, src_hbm, dst_hbm, out_hbm,
                idx_s, idx_d, rows, zbuf, acc, sem):
    cid = lax.axis_index("c")
    sid = lax.axis_index("s")
    w = cid * _NS + sid
    row0 = sid * _RPT

    _zero_fill(zbuf, _ZR, _D, _RPT // _ZR, acc, row0)
    plsc.subcore_barrier()

    pltpu.sync_copy(src_hbm.at[w], idx_s)
    pltpu.sync_copy(dst_hbm.at[w], idx_d)

    def body(j, carry):
        pltpu.async_copy(y_hbm.at[idx_s.at[j]], rows, sem).wait()
        pltpu.sync_copy(rows, acc.at[idx_d.at[j]], add=True)
        return carry

    lax.fori_loop(0, _T, body, 0)
    plsc.subcore_barrier()
    pltpu.sync_copy(acc.at[pl.ds(row0, _RPT)],
                    out_hbm.at[cid, pl.ds(row0, _RPT)])


_HIGH = lax.Precision.HIGHEST


def _tc_pre_body(cnt_ref, x_ref, w1_ref, y1_ref, dis_ref):
    deg = cnt_ref[0, :_N, :1] + cnt_ref[1, :_N, :1] + 1.0
    dis = lax.rsqrt(deg)
    h1 = jnp.dot(x_ref[...], w1_ref[...],
                 precision=_HIGH, preferred_element_type=jnp.float32)
    y1_ref[...] = h1 * dis
    dis_ref[...] = dis


def _tc_mid_body(acc_ref, y1_ref, dis_ref, b1_ref, g1_ref, be1_ref, w2_ref,
                 x1_ref, y2_ref):
    dis = dis_ref[...]
    s = (acc_ref[0, :_N] + acc_ref[1, :_N] + y1_ref[...]) * dis + b1_ref[...]
    mu = jnp.mean(s, axis=0, keepdims=True)
    var = jnp.mean((s - mu) ** 2, axis=0, keepdims=True)
    x1 = jnp.maximum(
        (s - mu) * lax.rsqrt(var + 1e-5) * g1_ref[...] + be1_ref[...], 0.0)
    x1_ref[...] = x1
    h2 = jnp.dot(x1, w2_ref[...],
                 precision=_HIGH, preferred_element_type=jnp.float32)
    y2_ref[...] = h2 * dis


def _tc_post_body(acc_ref, y2_ref, dis_ref, b2_ref, x1_ref, wf_ref, bf_ref,
                  out_ref):
    x2 = (acc_ref[0, :_N] + acc_ref[1, :_N] + y2_ref[...]) * dis_ref[...] + b2_ref[...]
    m = jnp.maximum(x1_ref[...], x2)
    out_ref[...] = jnp.dot(m, wf_ref[...],
                           precision=_HIGH,
                           preferred_element_type=jnp.float32) + bf_ref[...]


_f32 = jnp.float32

_tc_pre = pl.pallas_call(
    _tc_pre_body,
    out_shape=(jax.ShapeDtypeStruct((_N, _D), _f32),
               jax.ShapeDtypeStruct((_N, 1), _f32)),
)

_tc_mid = pl.pallas_call(
    _tc_mid_body,
    out_shape=(jax.ShapeDtypeStruct((_N, _D), _f32),
               jax.ShapeDtypeStruct((_N, _D), _f32)),
)

_tc_post = pl.pallas_call(
    _tc_post_body,
    out_shape=jax.ShapeDtypeStruct((_N, _D), _f32),
)


def kernel(node_feat, edge_index, W1, b1, gamma1, beta1, W2, b2, Wf, bf):
    src = edge_index[0].reshape(_NW, _T, _K)
    dst = edge_index[1].reshape(_NW, _T, _K)
    b1r, g1r, be1r = b1.reshape(1, _D), gamma1.reshape(1, _D), beta1.reshape(1, _D)
    b2r, bfr = b2.reshape(1, _D), bf.reshape(1, _D)

    cnt = _sc_degree(dst)
    y1, dis = _tc_pre(cnt, node_feat, W1)
    acc1 = _sc_scatter(y1, src, dst)
    x1, y2 = _tc_mid(acc1, y1, dis, b1r, g1r, be1r, W2)
    acc2 = _sc_scatter(y2, src, dst)
    return _tc_post(acc2, y2, dis, b2r, x1, Wf, bfr)
